# baseline (speedup 1.0000x reference)
import functools

import jax
import jax.numpy as jnp
from jax import lax
from jax.experimental import pallas as pl
from jax.experimental.pallas import tpu as pltpu
from jax.experimental.pallas import tpu_sc as plsc

_B, _S, _H = 4, 2048, 768
_EPS = 1e-5
_NC, _NS = 2, 16
_NW = _NC * _NS
_CHUNKS = (512, 512, 512, 512)
_RB = 128


def _gather_kernel(wp, ids_hbm, wemb_hbm, out_hbm, idx_v, buf_v,
                   semg, semg2, semw):
    wr = _B * wp
    wid = lax.axis_index("s") * _NC + lax.axis_index("c")
    base = wid * wr
    pltpu.sync_copy(ids_hbm.at[pl.ds(base, wr)], idx_v)
    half = wr // 2
    pltpu.async_copy(
        wemb_hbm.at[idx_v.at[pl.ds(0, half)]], buf_v.at[pl.ds(0, half)], semg)
    pltpu.async_copy(
        wemb_hbm.at[idx_v.at[pl.ds(half, half)]],
        buf_v.at[pl.ds(half, half)], semg2)
    pltpu.make_async_copy(
        wemb_hbm.at[idx_v.at[pl.ds(0, half)]],
        buf_v.at[pl.ds(0, half)], semg).wait()
    for b in range(2):
        pltpu.async_copy(
            buf_v.at[pl.ds(b * wp, wp)],
            out_hbm.at[b, pl.ds(wid * wp, wp)], semw)
    pltpu.make_async_copy(
        wemb_hbm.at[idx_v.at[pl.ds(half, half)]],
        buf_v.at[pl.ds(half, half)], semg2).wait()
    for b in range(2, _B):
        pltpu.async_copy(
            buf_v.at[pl.ds(b * wp, wp)],
            out_hbm.at[b, pl.ds(wid * wp, wp)], semw)
    for b in range(_B):
        pltpu.make_async_copy(
            buf_v.at[pl.ds(b * wp, wp)],
            out_hbm.at[b, pl.ds(wid * wp, wp)], semw).wait()


def _sc_gather(cs, ids_c, word_emb):
    wp = cs // _NW
    mesh = plsc.VectorSubcoreMesh(core_axis_name="c", subcore_axis_name="s")
    run = pl.kernel(
        functools.partial(_gather_kernel, wp),
        out_type=jax.ShapeDtypeStruct((_B, cs, _H), jnp.float32),
        mesh=mesh,
        scratch_types=[
            pltpu.VMEM((_B * wp,), jnp.int32),
            pltpu.VMEM((_B * wp, _H), jnp.float32),
            pltpu.SemaphoreType.DMA,
            pltpu.SemaphoreType.DMA,
            pltpu.SemaphoreType.DMA,
        ],
    )
    return run(ids_c, word_emb)


def _ln_body(buf_ref, g_ref, p_ref, t_ref, gamma_ref, beta_ref, o_ref):
    x = g_ref[0] + p_ref[...] + t_ref[0][None, :]
    mean = jnp.mean(x, axis=-1, keepdims=True)
    xc = x - mean
    var = jnp.mean(xc * xc, axis=-1, keepdims=True)
    o_ref[0] = xc * lax.rsqrt(var + _EPS) * gamma_ref[...] + beta_ref[...]


def _tc_ln(s0, cs, buf, g_c, pos_emb, type_row, gamma, beta):
    hb = cs // _RB
    p0 = s0 // _RB
    grid = (_B * hb,)
    data_specs = [
        pl.BlockSpec((1, _RB, _H), lambda i, _hb=hb: (i // _hb, i % _hb, 0)),
        pl.BlockSpec((_RB, _H), lambda i, _hb=hb, _p=p0: (_p + i % _hb, 0)),
        pl.BlockSpec((1, _H), lambda i: (0, 0)),
        pl.BlockSpec((_H,), lambda i: (0,)),
        pl.BlockSpec((_H,), lambda i: (0,)),
    ]
    out_spec = pl.BlockSpec(
        (1, _RB, _H), lambda i, _hb=hb, _p=p0: (i // _hb, _p + i % _hb, 0))
    out_shape = jax.ShapeDtypeStruct((_B, _S, _H), jnp.float32)
    if buf is None:
        return pl.pallas_call(
            functools.partial(_ln_body, None),
            grid=grid,
            in_specs=data_specs,
            out_specs=out_spec,
            out_shape=out_shape,
        )(g_c, pos_emb, type_row, gamma, beta)
    in_specs = [pl.BlockSpec((1, 8, 128), lambda i: (0, 0, 0))] + data_specs
    return pl.pallas_call(
        _ln_body,
        grid=grid,
        in_specs=in_specs,
        out_specs=out_spec,
        out_shape=out_shape,
        input_output_aliases={0: 0},
    )(buf, g_c, pos_emb, type_row, gamma, beta)


def kernel(input_ids, word_emb, pos_emb, type_emb, gamma, beta):
    type_row = type_emb.reshape(1, -1)[:, :_H]

    gathered = []
    s0 = 0
    for cs in _CHUNKS:
        wp = cs // _NW
        ids_c = (input_ids[:, s0:s0 + cs].reshape(_B, _NW, wp)
                 .transpose(1, 0, 2).reshape(-1))
        gathered.append(_sc_gather(cs, ids_c, word_emb))
        s0 += cs

    buf = None
    s0 = 0
    for c, cs in enumerate(_CHUNKS):
        buf = _tc_ln(s0, cs, buf, gathered[c], pos_emb, type_row, gamma, beta)
        s0 += cs
    return buf


# speedup vs baseline: 1.3284x; 1.3284x over previous
import functools

import jax
import jax.numpy as jnp
from jax import lax
from jax.experimental import pallas as pl
from jax.experimental.pallas import tpu as pltpu
from jax.experimental.pallas import tpu_sc as plsc

_B, _S, _H = 4, 2048, 768
_EPS = 1e-5
_NC, _NS = 2, 16
_NW = _NC * _NS
_CHUNKS = (512, 512, 512, 512)
_RB = 512


def _gather_kernel(wp, ids_hbm, wemb_hbm, out_hbm, idx_v, buf_v,
                   semg, semg2, semw):
    wr = _B * wp
    wid = lax.axis_index("s") * _NC + lax.axis_index("c")
    base = wid * wr
    pltpu.sync_copy(ids_hbm.at[pl.ds(base, wr)], idx_v)
    half = wr // 2
    pltpu.async_copy(
        wemb_hbm.at[idx_v.at[pl.ds(0, half)]], buf_v.at[pl.ds(0, half)], semg)
    pltpu.async_copy(
        wemb_hbm.at[idx_v.at[pl.ds(half, half)]],
        buf_v.at[pl.ds(half, half)], semg2)
    pltpu.make_async_copy(
        wemb_hbm.at[idx_v.at[pl.ds(0, half)]],
        buf_v.at[pl.ds(0, half)], semg).wait()
    for b in range(2):
        pltpu.async_copy(
            buf_v.at[pl.ds(b * wp, wp)],
            out_hbm.at[b, pl.ds(wid * wp, wp)], semw)
    pltpu.make_async_copy(
        wemb_hbm.at[idx_v.at[pl.ds(half, half)]],
        buf_v.at[pl.ds(half, half)], semg2).wait()
    for b in range(2, _B):
        pltpu.async_copy(
            buf_v.at[pl.ds(b * wp, wp)],
            out_hbm.at[b, pl.ds(wid * wp, wp)], semw)
    for b in range(_B):
        pltpu.make_async_copy(
            buf_v.at[pl.ds(b * wp, wp)],
            out_hbm.at[b, pl.ds(wid * wp, wp)], semw).wait()


def _sc_gather(cs, ids_c, word_emb):
    wp = cs // _NW
    mesh = plsc.VectorSubcoreMesh(core_axis_name="c", subcore_axis_name="s")
    run = pl.kernel(
        functools.partial(_gather_kernel, wp),
        out_type=jax.ShapeDtypeStruct((_B, cs, _H), jnp.float32),
        mesh=mesh,
        scratch_types=[
            pltpu.VMEM((_B * wp,), jnp.int32),
            pltpu.VMEM((_B * wp, _H), jnp.float32),
            pltpu.SemaphoreType.DMA,
            pltpu.SemaphoreType.DMA,
            pltpu.SemaphoreType.DMA,
        ],
    )
    return run(ids_c, word_emb)


def _ln_body(buf_ref, g_ref, p_ref, t_ref, gamma_ref, beta_ref, o_ref):
    x = g_ref[0] + p_ref[...] + t_ref[0][None, :]
    mean = jnp.mean(x, axis=-1, keepdims=True)
    xc = x - mean
    var = jnp.mean(xc * xc, axis=-1, keepdims=True)
    o_ref[0] = xc * lax.rsqrt(var + _EPS) * gamma_ref[...] + beta_ref[...]


def _tc_ln(s0, cs, buf, g_c, pos_emb, type_row, gamma, beta):
    hb = cs // _RB
    p0 = s0 // _RB
    grid = (_B * hb,)
    data_specs = [
        pl.BlockSpec((1, _RB, _H), lambda i, _hb=hb: (i // _hb, i % _hb, 0)),
        pl.BlockSpec((_RB, _H), lambda i, _hb=hb, _p=p0: (_p + i % _hb, 0)),
        pl.BlockSpec((1, _H), lambda i: (0, 0)),
        pl.BlockSpec((_H,), lambda i: (0,)),
        pl.BlockSpec((_H,), lambda i: (0,)),
    ]
    out_spec = pl.BlockSpec(
        (1, _RB, _H), lambda i, _hb=hb, _p=p0: (i // _hb, _p + i % _hb, 0))
    out_shape = jax.ShapeDtypeStruct((_B, _S, _H), jnp.float32)
    if buf is None:
        return pl.pallas_call(
            functools.partial(_ln_body, None),
            grid=grid,
            in_specs=data_specs,
            out_specs=out_spec,
            out_shape=out_shape,
        )(g_c, pos_emb, type_row, gamma, beta)
    in_specs = [pl.BlockSpec((1, 8, 128), lambda i: (0, 0, 0))] + data_specs
    return pl.pallas_call(
        _ln_body,
        grid=grid,
        in_specs=in_specs,
        out_specs=out_spec,
        out_shape=out_shape,
        input_output_aliases={0: 0},
    )(buf, g_c, pos_emb, type_row, gamma, beta)


def kernel(input_ids, word_emb, pos_emb, type_emb, gamma, beta):
    type_row = type_emb.reshape(1, -1)[:, :_H]

    gathered = []
    s0 = 0
    for cs in _CHUNKS:
        wp = cs // _NW
        ids_c = (input_ids[:, s0:s0 + cs].reshape(_B, _NW, wp)
                 .transpose(1, 0, 2).reshape(-1))
        gathered.append(_sc_gather(cs, ids_c, word_emb))
        s0 += cs

    buf = None
    s0 = 0
    for c, cs in enumerate(_CHUNKS):
        buf = _tc_ln(s0, cs, buf, gathered[c], pos_emb, type_row, gamma, beta)
        s0 += cs
    return buf


# speedup vs baseline: 1.3761x; 1.0359x over previous
import functools

import jax
import jax.numpy as jnp
from jax import lax
from jax.experimental import pallas as pl
from jax.experimental.pallas import tpu as pltpu
from jax.experimental.pallas import tpu_sc as plsc

_B, _S, _H = 4, 2048, 768
_EPS = 1e-5
_NC, _NS = 2, 16
_NW = _NC * _NS
_CHUNKS = (512, 512, 512, 512)
_RB = 512
_BB = 2


def _gather_kernel(wp, ids_hbm, wemb_hbm, out_hbm, idx_v, buf_v,
                   semg, semg2, semw):
    wr = _B * wp
    wid = lax.axis_index("s") * _NC + lax.axis_index("c")
    base = wid * wr
    pltpu.sync_copy(ids_hbm.at[pl.ds(base, wr)], idx_v)
    half = wr // 2
    pltpu.async_copy(
        wemb_hbm.at[idx_v.at[pl.ds(0, half)]], buf_v.at[pl.ds(0, half)], semg)
    pltpu.async_copy(
        wemb_hbm.at[idx_v.at[pl.ds(half, half)]],
        buf_v.at[pl.ds(half, half)], semg2)
    pltpu.make_async_copy(
        wemb_hbm.at[idx_v.at[pl.ds(0, half)]],
        buf_v.at[pl.ds(0, half)], semg).wait()
    for b in range(2):
        pltpu.async_copy(
            buf_v.at[pl.ds(b * wp, wp)],
            out_hbm.at[b, pl.ds(wid * wp, wp)], semw)
    pltpu.make_async_copy(
        wemb_hbm.at[idx_v.at[pl.ds(half, half)]],
        buf_v.at[pl.ds(half, half)], semg2).wait()
    for b in range(2, _B):
        pltpu.async_copy(
            buf_v.at[pl.ds(b * wp, wp)],
            out_hbm.at[b, pl.ds(wid * wp, wp)], semw)
    for b in range(_B):
        pltpu.make_async_copy(
            buf_v.at[pl.ds(b * wp, wp)],
            out_hbm.at[b, pl.ds(wid * wp, wp)], semw).wait()


def _sc_gather(cs, ids_c, word_emb):
    wp = cs // _NW
    mesh = plsc.VectorSubcoreMesh(core_axis_name="c", subcore_axis_name="s")
    run = pl.kernel(
        functools.partial(_gather_kernel, wp),
        out_type=jax.ShapeDtypeStruct((_B, cs, _H), jnp.float32),
        mesh=mesh,
        scratch_types=[
            pltpu.VMEM((_B * wp,), jnp.int32),
            pltpu.VMEM((_B * wp, _H), jnp.float32),
            pltpu.SemaphoreType.DMA,
            pltpu.SemaphoreType.DMA,
            pltpu.SemaphoreType.DMA,
        ],
    )
    return run(ids_c, word_emb)


def _ln_body(buf_ref, g_ref, p_ref, t_ref, gamma_ref, beta_ref, o_ref):
    x = g_ref[...] + p_ref[...][None] + t_ref[0][None, None, :]
    mean = jnp.mean(x, axis=-1, keepdims=True)
    xc = x - mean
    var = jnp.mean(xc * xc, axis=-1, keepdims=True)
    o_ref[...] = xc * lax.rsqrt(var + _EPS) * gamma_ref[...] + beta_ref[...]


def _tc_ln(s0, cs, buf, g_c, pos_emb, type_row, gamma, beta):
    hb = cs // _RB
    p0 = s0 // _RB
    grid = (_B // _BB * hb,)
    data_specs = [
        pl.BlockSpec((_BB, _RB, _H), lambda i, _hb=hb: (i // _hb, i % _hb, 0)),
        pl.BlockSpec((_RB, _H), lambda i, _hb=hb, _p=p0: (_p + i % _hb, 0)),
        pl.BlockSpec((1, _H), lambda i: (0, 0)),
        pl.BlockSpec((_H,), lambda i: (0,)),
        pl.BlockSpec((_H,), lambda i: (0,)),
    ]
    out_spec = pl.BlockSpec(
        (_BB, _RB, _H), lambda i, _hb=hb, _p=p0: (i // _hb, _p + i % _hb, 0))
    out_shape = jax.ShapeDtypeStruct((_B, _S, _H), jnp.float32)
    if buf is None:
        return pl.pallas_call(
            functools.partial(_ln_body, None),
            grid=grid,
            in_specs=data_specs,
            out_specs=out_spec,
            out_shape=out_shape,
        )(g_c, pos_emb, type_row, gamma, beta)
    in_specs = [pl.BlockSpec((1, 8, 128), lambda i: (0, 0, 0))] + data_specs
    return pl.pallas_call(
        _ln_body,
        grid=grid,
        in_specs=in_specs,
        out_specs=out_spec,
        out_shape=out_shape,
        input_output_aliases={0: 0},
    )(buf, g_c, pos_emb, type_row, gamma, beta)


def kernel(input_ids, word_emb, pos_emb, type_emb, gamma, beta):
    type_row = type_emb.reshape(1, -1)[:, :_H]

    gathered = []
    s0 = 0
    for cs in _CHUNKS:
        wp = cs // _NW
        ids_c = (input_ids[:, s0:s0 + cs].reshape(_B, _NW, wp)
                 .transpose(1, 0, 2).reshape(-1))
        gathered.append(_sc_gather(cs, ids_c, word_emb))
        s0 += cs

    buf = None
    s0 = 0
    for c, cs in enumerate(_CHUNKS):
        buf = _tc_ln(s0, cs, buf, gathered[c], pos_emb, type_row, gamma, beta)
        s0 += cs
    return buf
